# baseline (device time: 17588 ns/iter reference)
import jax
import jax.numpy as jnp
from jax import lax
from jax.experimental import pallas as pl
from jax.experimental.pallas import tpu as pltpu

N_DEV = 4


def kernel(partial, gamma):
    _, m, d = partial.shape
    m_out = m // N_DEV
    mh = m_out // 2

    def body(p_ref, g_ref, out_ref, s1, s2, pre, rbuf, send_sems, recv_sems):
        my = lax.axis_index("i")
        left = lax.rem(my + N_DEV - 1, N_DEV)
        right = lax.rem(my + 1, N_DEV)
        diag = lax.rem(my + 2, N_DEV)

        barrier_sem = pltpu.get_barrier_semaphore()
        for nbr in (left, right):
            pl.semaphore_signal(
                barrier_sem, inc=1,
                device_id=(nbr,), device_id_type=pl.DeviceIdType.MESH,
            )

        def contrib(c, half):
            return p_ref[0, pl.ds(c * m_out + half * mh, mh), :].astype(
                jnp.bfloat16
            )

        def contrib32(c, half):
            return p_ref[0, pl.ds(c * m_out + half * mh, mh), :]

        s1[0] = contrib(diag, 0)
        s1[2] = contrib(diag, 1)
        s1[1] = contrib(right, 1)
        s1[3] = contrib(left, 0)
        pl.semaphore_wait(barrier_sem, 2)

        def copy(src_slot, dst_slot, sem_idx, tgt):
            return pltpu.make_async_remote_copy(
                src_ref=src_slot,
                dst_ref=rbuf.at[dst_slot],
                send_sem=send_sems.at[sem_idx],
                recv_sem=recv_sems.at[dst_slot],
                device_id=(tgt,),
                device_id_type=pl.DeviceIdType.MESH,
            )

        fwd_r = copy(s1.at[0], 0, 0, right)
        fwd_l = copy(s1.at[2], 3, 2, left)
        dir_r = copy(s1.at[1], 1, 1, right)
        dir_l = copy(s1.at[3], 4, 3, left)
        fwd_r.start()
        fwd_l.start()
        dir_r.start()
        dir_l.start()

        pre[0] = contrib(right, 0)
        pre[1] = contrib(left, 1)

        fwd_r.wait_recv()
        s2[0] = rbuf[0] + pre[0]
        comb_r = copy(s2.at[0], 2, 4, right)
        comb_r.start()
        fwd_l.wait_recv()
        s2[1] = rbuf[3] + pre[1]
        comb_l = copy(s2.at[1], 5, 5, left)
        comb_l.start()

        dir_l.wait_recv()
        part_a = contrib32(my, 0) + rbuf[4].astype(jnp.float32)
        dir_r.wait_recv()
        part_b = contrib32(my, 1) + rbuf[1].astype(jnp.float32)

        def finish(half, part, comb_slot, comb_rdma):
            comb_rdma.wait_recv()
            y = part + rbuf[comb_slot].astype(jnp.float32)
            rms = jnp.sqrt(jnp.mean(y * y, axis=-1, keepdims=True) + 1e-6)
            out_ref[pl.ds(half * mh, mh), :] = y / rms * g_ref[...]

        finish(0, part_a, 2, comb_r)
        finish(1, part_b, 5, comb_l)

        for r in (fwd_r, fwd_l, dir_r, dir_l, comb_r, comb_l):
            r.wait_send()

    return pl.pallas_call(
        body,
        out_shape=jax.ShapeDtypeStruct((m_out, d), jnp.float32),
        in_specs=[
            pl.BlockSpec(memory_space=pltpu.VMEM),
            pl.BlockSpec(memory_space=pltpu.VMEM),
        ],
        out_specs=pl.BlockSpec(memory_space=pltpu.VMEM),
        scratch_shapes=[
            pltpu.VMEM((4, mh, d), jnp.bfloat16),
            pltpu.VMEM((2, mh, d), jnp.bfloat16),
            pltpu.VMEM((2, mh, d), jnp.bfloat16),
            pltpu.VMEM((6, mh, d), jnp.bfloat16),
            pltpu.SemaphoreType.DMA((6,)),
            pltpu.SemaphoreType.DMA((6,)),
        ],
        compiler_params=pltpu.CompilerParams(collective_id=0),
    )(partial, gamma)


# device time: 16570 ns/iter; 1.0614x vs baseline; 1.0614x over previous
import jax
import jax.numpy as jnp
from jax import lax
from jax.experimental import pallas as pl
from jax.experimental.pallas import tpu as pltpu

N_DEV = 4


def kernel(partial, gamma):
    _, m, d = partial.shape
    m_out = m // N_DEV
    mh = m_out // 2
    partial = jnp.reshape(partial, (m, d))

    def body(p_ref, g_ref, out_ref, s1, s2, pre, rbuf, send_sems, recv_sems):
        my = lax.axis_index("i")
        left = lax.rem(my + N_DEV - 1, N_DEV)
        right = lax.rem(my + 1, N_DEV)
        diag = lax.rem(my + 2, N_DEV)

        barrier_sem = pltpu.get_barrier_semaphore()
        for nbr in (left, right):
            pl.semaphore_signal(
                barrier_sem, inc=1,
                device_id=(nbr,), device_id_type=pl.DeviceIdType.MESH,
            )

        def contrib(c, half):
            return p_ref[pl.ds(c * m_out + half * mh, mh), :].astype(
                jnp.bfloat16
            )

        def contrib32(c, half):
            return p_ref[pl.ds(c * m_out + half * mh, mh), :]

        s1[0] = contrib(diag, 0)
        s1[2] = contrib(diag, 1)
        s1[1] = contrib(right, 1)
        s1[3] = contrib(left, 0)
        pl.semaphore_wait(barrier_sem, 2)

        def copy(src_slot, dst_slot, sem_idx, tgt):
            return pltpu.make_async_remote_copy(
                src_ref=src_slot,
                dst_ref=rbuf.at[dst_slot],
                send_sem=send_sems.at[sem_idx],
                recv_sem=recv_sems.at[dst_slot],
                device_id=(tgt,),
                device_id_type=pl.DeviceIdType.MESH,
            )

        fwd_r = copy(s1.at[0], 0, 0, right)
        fwd_l = copy(s1.at[2], 3, 2, left)
        dir_r = copy(s1.at[1], 1, 1, right)
        dir_l = copy(s1.at[3], 4, 3, left)
        fwd_r.start()
        fwd_l.start()
        dir_r.start()
        dir_l.start()

        pre[0] = contrib(right, 0)
        pre[1] = contrib(left, 1)

        fwd_r.wait_recv()
        s2[0] = rbuf[0] + pre[0]
        comb_r = copy(s2.at[0], 2, 4, right)
        comb_r.start()
        fwd_l.wait_recv()
        s2[1] = rbuf[3] + pre[1]
        comb_l = copy(s2.at[1], 5, 5, left)
        comb_l.start()

        dir_l.wait_recv()
        part_a = contrib32(my, 0) + rbuf[4].astype(jnp.float32)
        dir_r.wait_recv()
        part_b = contrib32(my, 1) + rbuf[1].astype(jnp.float32)

        def finish(half, part, comb_slot, comb_rdma):
            comb_rdma.wait_recv()
            y = part + rbuf[comb_slot].astype(jnp.float32)
            rms = jnp.sqrt(jnp.mean(y * y, axis=-1, keepdims=True) + 1e-6)
            out_ref[pl.ds(half * mh, mh), :] = y / rms * g_ref[...]

        finish(0, part_a, 2, comb_r)
        finish(1, part_b, 5, comb_l)

        for r in (fwd_r, fwd_l, dir_r, dir_l, comb_r, comb_l):
            r.wait_send()

    return pl.pallas_call(
        body,
        out_shape=jax.ShapeDtypeStruct((m_out, d), jnp.float32),
        in_specs=[
            pl.BlockSpec(memory_space=pltpu.VMEM),
            pl.BlockSpec(memory_space=pltpu.VMEM),
        ],
        out_specs=pl.BlockSpec(memory_space=pltpu.VMEM),
        scratch_shapes=[
            pltpu.VMEM((4, mh, d), jnp.bfloat16),
            pltpu.VMEM((2, mh, d), jnp.bfloat16),
            pltpu.VMEM((2, mh, d), jnp.bfloat16),
            pltpu.VMEM((6, mh, d), jnp.bfloat16),
            pltpu.SemaphoreType.DMA((6,)),
            pltpu.SemaphoreType.DMA((6,)),
        ],
        compiler_params=pltpu.CompilerParams(collective_id=0),
    )(partial, gamma)
